# Initial kernel scaffold; baseline (speedup 1.0000x reference)
import jax
import jax.numpy as jnp
from jax.experimental import pallas as pl


def kernel(input, KNN_idx, KNN_xyz, W1, b1, W2, b2):
    raise NotImplementedError("write your pallas kernel here")


# speedup vs baseline: 1.3805x; 1.3805x over previous
import functools

import jax
import jax.numpy as jnp
from jax import lax
from jax.experimental import pallas as pl
from jax.experimental.pallas import tpu as pltpu
from jax.experimental.pallas import tpu_sc as plsc

N_NODES = 50000
KNN_NUM = 8
EF_DIM = 128
N_GATHER = N_NODES * KNN_NUM

NC, NS = 2, 16
NW = NC * NS
CH = 128
NCH = 100
PER_W = CH * NCH
PAD_ROWS = NW * PER_W

PRE_BLK = 2000
TAIL_ROWS = 8000
TAIL_PTS = TAIL_ROWS // KNN_NUM


def _pre_body(x_ref, w_ref, b_ref, o_ref):
    o_ref[...] = (
        jnp.dot(x_ref[...], w_ref[...], preferred_element_type=jnp.float32)
        + b_ref[...]
    )


def _tail_body(g_ref, xyz_ref, wx_ref, w2_ref, b2_ref, o_ref):
    pre = g_ref[...] + jnp.dot(
        xyz_ref[...], wx_ref[...], preferred_element_type=jnp.float32
    )
    act = jnp.where(pre >= 0, pre, 0.01 * pre)
    h2 = (
        jnp.dot(act, w2_ref[...], preferred_element_type=jnp.float32)
        + b2_ref[...]
    )
    o_ref[...] = jnp.max(h2.reshape(TAIL_PTS, KNN_NUM, EF_DIM), axis=1)


def _sc_gather_body(h_hbm, idx_hbm, out_hbm, idx_v, rows_a, rows_b, sem_a, sem_b):
    wid = lax.axis_index("s") * NC + lax.axis_index("c")
    base_ch = wid * NCH
    pltpu.sync_copy(idx_hbm.at[wid], idx_v)

    def body(i, carry):
        c0 = 2 * i
        c1 = c0 + 1
        d0 = pltpu.async_copy(h_hbm.at[idx_v.at[c0]], rows_a, sem_a)
        d1 = pltpu.async_copy(h_hbm.at[idx_v.at[c1]], rows_b, sem_b)
        d0.wait()
        pltpu.sync_copy(rows_a, out_hbm.at[pl.ds((base_ch + c0) * CH, CH)])
        d1.wait()
        pltpu.sync_copy(rows_b, out_hbm.at[pl.ds((base_ch + c1) * CH, CH)])
        return carry

    lax.fori_loop(0, NCH // 2, body, 0)


@functools.lru_cache(maxsize=1)
def _sc_gather():
    return pl.kernel(
        _sc_gather_body,
        out_type=jax.ShapeDtypeStruct((PAD_ROWS, EF_DIM), jnp.float32),
        mesh=plsc.VectorSubcoreMesh(
            core_axis_name="c", subcore_axis_name="s", num_cores=NC, num_subcores=NS
        ),
        scratch_types=[
            pltpu.VMEM((NCH, CH), jnp.int32),
            pltpu.VMEM((CH, EF_DIM), jnp.float32),
            pltpu.VMEM((CH, EF_DIM), jnp.float32),
            pltpu.SemaphoreType.DMA,
            pltpu.SemaphoreType.DMA,
        ],
    )


def kernel(input, KNN_idx, KNN_xyz, W1, b1, W2, b2):
    idx = KNN_idx.astype(jnp.int32)
    w1f_t = W1[:, :EF_DIM].T
    w1x_t = W1[:, EF_DIM:].T
    w2_t = W2.T

    h = pl.pallas_call(
        _pre_body,
        grid=(N_NODES // PRE_BLK,),
        in_specs=[
            pl.BlockSpec((PRE_BLK, EF_DIM), lambda i: (i, 0)),
            pl.BlockSpec((EF_DIM, EF_DIM), lambda i: (0, 0)),
            pl.BlockSpec((1, EF_DIM), lambda i: (0, 0)),
        ],
        out_specs=pl.BlockSpec((PRE_BLK, EF_DIM), lambda i: (i, 0)),
        out_shape=jax.ShapeDtypeStruct((N_NODES, EF_DIM), jnp.float32),
    )(input, w1f_t, b1.reshape(1, EF_DIM))

    idx_pad = jnp.concatenate(
        [idx, jnp.zeros((PAD_ROWS - N_GATHER,), jnp.int32)]
    ).reshape(NW, NCH, CH)
    g = _sc_gather()(h, idx_pad)

    out = pl.pallas_call(
        _tail_body,
        grid=(N_GATHER // TAIL_ROWS,),
        in_specs=[
            pl.BlockSpec((TAIL_ROWS, EF_DIM), lambda i: (i, 0)),
            pl.BlockSpec((TAIL_ROWS, 3), lambda i: (i, 0)),
            pl.BlockSpec((3, EF_DIM), lambda i: (0, 0)),
            pl.BlockSpec((EF_DIM, EF_DIM), lambda i: (0, 0)),
            pl.BlockSpec((1, EF_DIM), lambda i: (0, 0)),
        ],
        out_specs=pl.BlockSpec((TAIL_PTS, EF_DIM), lambda i: (i, 0)),
        out_shape=jax.ShapeDtypeStruct((N_NODES, EF_DIM), jnp.float32),
    )(g, KNN_xyz, w1x_t, w2_t, b2.reshape(1, EF_DIM))
    return out


# speedup vs baseline: 1.4564x; 1.0550x over previous
import functools

import jax
import jax.numpy as jnp
from jax import lax
from jax.experimental import pallas as pl
from jax.experimental.pallas import tpu as pltpu
from jax.experimental.pallas import tpu_sc as plsc

N_NODES = 50000
KNN_NUM = 8
EF_DIM = 128
N_GATHER = N_NODES * KNN_NUM

NC, NS = 2, 16
NW = NC * NS
CH = 128
NCH = 100
PER_W = CH * NCH
PAD_ROWS = NW * PER_W

PRE_BLK = 2000
TAIL_ROWS = 8000
TAIL_PTS = TAIL_ROWS // KNN_NUM


def _pre_body(x_ref, w_ref, b_ref, o_ref):
    o_ref[...] = (
        jnp.dot(x_ref[...], w_ref[...], preferred_element_type=jnp.float32)
        + b_ref[...]
    )


def _tail_body(g_ref, xyz_ref, wx_ref, w2_ref, b2_ref, o_ref):
    pre = g_ref[...] + jnp.dot(
        xyz_ref[...], wx_ref[...], preferred_element_type=jnp.float32
    )
    act = jnp.where(pre >= 0, pre, 0.01 * pre)
    h2 = (
        jnp.dot(act, w2_ref[...], preferred_element_type=jnp.float32)
        + b2_ref[...]
    )
    o_ref[...] = jnp.max(h2.reshape(TAIL_PTS, KNN_NUM, EF_DIM), axis=1)


NBUF = 5
GWIN = 3


def _sc_gather_body(h_hbm, idx_hbm, out_hbm, idx_v, *scr):
    rows = scr[0:NBUF]
    gsem = scr[NBUF : 2 * NBUF]
    ssem = scr[2 * NBUF : 3 * NBUF]
    wid = lax.axis_index("s") * NC + lax.axis_index("c")
    base_ch = wid * NCH
    pltpu.sync_copy(idx_hbm.at[wid], idx_v)

    def out_slice(c):
        return out_hbm.at[pl.ds((base_ch + c) * CH, CH)]

    def fire_gather(c, b):
        pltpu.async_copy(h_hbm.at[idx_v.at[c]], rows[b], gsem[b])

    def wait_gather(c, b):
        pltpu.make_async_copy(h_hbm.at[idx_v.at[c]], rows[b], gsem[b]).wait()

    def fire_store(c, b):
        pltpu.async_copy(rows[b], out_slice(c), ssem[b])

    def wait_store(c, b):
        pltpu.make_async_copy(rows[b], out_slice(c), ssem[b]).wait()

    def visit(c, b, do_store_wait, do_gather_fire):
        wait_gather(c, b)
        fire_store(c, b)
        b2 = (b + GWIN) % NBUF
        if do_store_wait:
            wait_store(c + GWIN - NBUF, b2)
        if do_gather_fire:
            fire_gather(c + GWIN, b2)

    for c in range(GWIN):
        fire_gather(c, c)
    for b in range(NBUF):
        visit(b, b, do_store_wait=(b + GWIN - NBUF >= 0), do_gather_fire=True)

    def round_body(j, carry):
        for b in range(NBUF):
            visit(NBUF * j + b, b, do_store_wait=True, do_gather_fire=True)
        return carry

    lax.fori_loop(1, NCH // NBUF - 1, round_body, 0)

    for b in range(NBUF):
        c = NCH - NBUF + b
        visit(c, b, do_store_wait=True, do_gather_fire=(c + GWIN < NCH))
    for c in range(NCH + GWIN - NBUF, NCH):
        wait_store(c, c % NBUF)


@functools.lru_cache(maxsize=1)
def _sc_gather():
    return pl.kernel(
        _sc_gather_body,
        out_type=jax.ShapeDtypeStruct((PAD_ROWS, EF_DIM), jnp.float32),
        mesh=plsc.VectorSubcoreMesh(
            core_axis_name="c", subcore_axis_name="s", num_cores=NC, num_subcores=NS
        ),
        scratch_types=(
            [pltpu.VMEM((NCH, CH), jnp.int32)]
            + [pltpu.VMEM((CH, EF_DIM), jnp.float32) for _ in range(NBUF)]
            + [pltpu.SemaphoreType.DMA for _ in range(2 * NBUF)]
        ),
    )


def kernel(input, KNN_idx, KNN_xyz, W1, b1, W2, b2):
    idx = KNN_idx.astype(jnp.int32)
    w1f_t = W1[:, :EF_DIM].T
    w1x_t = W1[:, EF_DIM:].T
    w2_t = W2.T

    h = pl.pallas_call(
        _pre_body,
        grid=(N_NODES // PRE_BLK,),
        in_specs=[
            pl.BlockSpec((PRE_BLK, EF_DIM), lambda i: (i, 0)),
            pl.BlockSpec((EF_DIM, EF_DIM), lambda i: (0, 0)),
            pl.BlockSpec((1, EF_DIM), lambda i: (0, 0)),
        ],
        out_specs=pl.BlockSpec((PRE_BLK, EF_DIM), lambda i: (i, 0)),
        out_shape=jax.ShapeDtypeStruct((N_NODES, EF_DIM), jnp.float32),
    )(input, w1f_t, b1.reshape(1, EF_DIM))

    idx_pad = jnp.concatenate(
        [idx, jnp.zeros((PAD_ROWS - N_GATHER,), jnp.int32)]
    ).reshape(NW, NCH, CH)
    g = _sc_gather()(h, idx_pad)

    out = pl.pallas_call(
        _tail_body,
        grid=(N_GATHER // TAIL_ROWS,),
        in_specs=[
            pl.BlockSpec((TAIL_ROWS, EF_DIM), lambda i: (i, 0)),
            pl.BlockSpec((TAIL_ROWS, 3), lambda i: (i, 0)),
            pl.BlockSpec((3, EF_DIM), lambda i: (0, 0)),
            pl.BlockSpec((EF_DIM, EF_DIM), lambda i: (0, 0)),
            pl.BlockSpec((1, EF_DIM), lambda i: (0, 0)),
        ],
        out_specs=pl.BlockSpec((TAIL_PTS, EF_DIM), lambda i: (i, 0)),
        out_shape=jax.ShapeDtypeStruct((N_NODES, EF_DIM), jnp.float32),
    )(g, KNN_xyz, w1x_t, w2_t, b2.reshape(1, EF_DIM))
    return out


# speedup vs baseline: 1.4572x; 1.0005x over previous
import functools

import jax
import jax.numpy as jnp
from jax import lax
from jax.experimental import pallas as pl
from jax.experimental.pallas import tpu as pltpu
from jax.experimental.pallas import tpu_sc as plsc

N_NODES = 50000
KNN_NUM = 8
EF_DIM = 128
N_GATHER = N_NODES * KNN_NUM

NC, NS = 2, 16
NW = NC * NS
CH = 128
NCH = 100
PER_W = CH * NCH
PAD_ROWS = NW * PER_W

PRE_BLK = 2000
TAIL_ROWS = 8000
TAIL_PTS = TAIL_ROWS // KNN_NUM


def _pre_body(x_ref, w_ref, b_ref, o_ref):
    o_ref[...] = (
        jnp.dot(x_ref[...], w_ref[...], preferred_element_type=jnp.float32)
        + b_ref[...]
    )


def _tail_body(g_ref, xyz_ref, wx_ref, w2_ref, b2_ref, o_ref):
    pre = g_ref[...] + jnp.dot(
        xyz_ref[...], wx_ref[...], preferred_element_type=jnp.float32
    )
    act = jnp.where(pre >= 0, pre, 0.01 * pre)
    h2 = (
        jnp.dot(act, w2_ref[...], preferred_element_type=jnp.float32)
        + b2_ref[...]
    )
    o_ref[...] = jnp.max(h2.reshape(TAIL_PTS, KNN_NUM, EF_DIM), axis=1)


NBUF = 5
GWIN = 3


def _sc_gather_body(h_hbm, idx_hbm, out_hbm, idx_v, *scr):
    rows = scr[0:NBUF]
    gsem = scr[NBUF : 2 * NBUF]
    ssem = scr[2 * NBUF : 3 * NBUF]
    wid = lax.axis_index("c") * NS + lax.axis_index("s")
    base_ch = wid * NCH
    pltpu.sync_copy(idx_hbm.at[wid], idx_v)

    def out_slice(c):
        return out_hbm.at[pl.ds((base_ch + c) * CH, CH)]

    def fire_gather(c, b):
        pltpu.async_copy(h_hbm.at[idx_v.at[c]], rows[b], gsem[b])

    def wait_gather(c, b):
        pltpu.make_async_copy(h_hbm.at[idx_v.at[c]], rows[b], gsem[b]).wait()

    def fire_store(c, b):
        pltpu.async_copy(rows[b], out_slice(c), ssem[b])

    def wait_store(c, b):
        pltpu.make_async_copy(rows[b], out_slice(c), ssem[b]).wait()

    def visit(c, b, do_store_wait, do_gather_fire):
        wait_gather(c, b)
        fire_store(c, b)
        b2 = (b + GWIN) % NBUF
        if do_store_wait:
            wait_store(c + GWIN - NBUF, b2)
        if do_gather_fire:
            fire_gather(c + GWIN, b2)

    for c in range(GWIN):
        fire_gather(c, c)
    for b in range(NBUF):
        visit(b, b, do_store_wait=(b + GWIN - NBUF >= 0), do_gather_fire=True)

    def round_body(j, carry):
        for b in range(NBUF):
            visit(NBUF * j + b, b, do_store_wait=True, do_gather_fire=True)
        return carry

    lax.fori_loop(1, NCH // NBUF - 1, round_body, 0)

    for b in range(NBUF):
        c = NCH - NBUF + b
        visit(c, b, do_store_wait=True, do_gather_fire=(c + GWIN < NCH))
    for c in range(NCH + GWIN - NBUF, NCH):
        wait_store(c, c % NBUF)


@functools.lru_cache(maxsize=1)
def _sc_gather():
    return pl.kernel(
        _sc_gather_body,
        out_type=jax.ShapeDtypeStruct((PAD_ROWS, EF_DIM), jnp.float32),
        mesh=plsc.VectorSubcoreMesh(
            core_axis_name="c", subcore_axis_name="s", num_cores=NC, num_subcores=NS
        ),
        scratch_types=(
            [pltpu.VMEM((NCH, CH), jnp.int32)]
            + [pltpu.VMEM((CH, EF_DIM), jnp.float32) for _ in range(NBUF)]
            + [pltpu.SemaphoreType.DMA for _ in range(2 * NBUF)]
        ),
    )


def kernel(input, KNN_idx, KNN_xyz, W1, b1, W2, b2):
    idx = KNN_idx.astype(jnp.int32)
    w1f_t = W1[:, :EF_DIM].T
    w1x_t = W1[:, EF_DIM:].T
    w2_t = W2.T

    h = pl.pallas_call(
        _pre_body,
        grid=(N_NODES // PRE_BLK,),
        in_specs=[
            pl.BlockSpec((PRE_BLK, EF_DIM), lambda i: (i, 0)),
            pl.BlockSpec((EF_DIM, EF_DIM), lambda i: (0, 0)),
            pl.BlockSpec((1, EF_DIM), lambda i: (0, 0)),
        ],
        out_specs=pl.BlockSpec((PRE_BLK, EF_DIM), lambda i: (i, 0)),
        out_shape=jax.ShapeDtypeStruct((N_NODES, EF_DIM), jnp.float32),
    )(input, w1f_t, b1.reshape(1, EF_DIM))
    idx_pad = jnp.concatenate(
        [idx, jnp.zeros((PAD_ROWS - N_GATHER,), jnp.int32)]
    ).reshape(NW, NCH, CH)
    g = _sc_gather()(h, idx_pad)

    out = pl.pallas_call(
        _tail_body,
        grid=(N_GATHER // TAIL_ROWS,),
        in_specs=[
            pl.BlockSpec((TAIL_ROWS, EF_DIM), lambda i: (i, 0)),
            pl.BlockSpec((TAIL_ROWS, 3), lambda i: (i, 0)),
            pl.BlockSpec((3, EF_DIM), lambda i: (0, 0)),
            pl.BlockSpec((EF_DIM, EF_DIM), lambda i: (0, 0)),
            pl.BlockSpec((1, EF_DIM), lambda i: (0, 0)),
        ],
        out_specs=pl.BlockSpec((TAIL_PTS, EF_DIM), lambda i: (i, 0)),
        out_shape=jax.ShapeDtypeStruct((N_NODES, EF_DIM), jnp.float32),
    )(g, KNN_xyz, w1x_t, w2_t, b2.reshape(1, EF_DIM))
    return out
